# baseline (speedup 1.0000x reference)
import functools

import jax
import jax.numpy as jnp
from jax import lax
from jax.experimental import pallas as pl
from jax.experimental.pallas import tpu as pltpu
from jax.experimental.pallas import tpu_sc as plsc

_K = 128


def _mesh():
    return plsc.VectorSubcoreMesh(core_axis_name="c", subcore_axis_name="s")


def _row_split(n, ns):
    rpt = n // ns
    assert rpt * ns == n
    sz = -((rpt + 7) // -8) * 8
    assert ((ns - 1) * rpt // 8) * 8 + sz == n
    return rpt, sz


@functools.lru_cache(maxsize=None)
def _deg_call(n, cpw, nc, ns):
    nw = nc * ns
    rpt, sz = _row_split(n, ns)
    nb = 4
    assert cpw % nb == 0 and cpw > nb

    @functools.partial(
        pl.kernel,
        out_type=jax.ShapeDtypeStruct((nc, n, 16), jnp.float32),
        mesh=_mesh(),
        compiler_params=pltpu.CompilerParams(use_tc_tiling_on_sc=False),
        scratch_types=[
            pltpu.VMEM((cpw, _K), jnp.int32),
            pltpu.VMEM((_K, 16), jnp.float32),
            pltpu.VMEM_SHARED((n + 8, 16), jnp.float32),
        ] + [pltpu.SemaphoreType.DMA] * nb,
    )
    def deg_kernel(dst_hbm, ones_hbm, zeros_hbm, out_hbm,
                   idx_v, ones_v, acc_sh, *sems):
        c_ax = lax.axis_index("c")
        s_ax = lax.axis_index("s")
        w = c_ax * ns + s_ax
        start = pl.multiple_of(s_ax * rpt // 8 * 8, 8)
        pltpu.sync_copy(ones_hbm, ones_v)
        pltpu.sync_copy(dst_hbm.at[pl.ds(w * cpw, cpw)], idx_v)
        pltpu.sync_copy(zeros_hbm.at[pl.ds(start, sz)],
                        acc_sh.at[pl.ds(start, sz)])
        plsc.subcore_barrier()
        for b in range(nb):
            pltpu.async_copy(ones_v, acc_sh.at[idx_v.at[b]], sems[b],
                             add=True)

        def round_body(r, carry):
            for b in range(nb):
                c = nb + r * nb + b
                pltpu.make_async_copy(ones_v, acc_sh.at[idx_v.at[c - nb]],
                                      sems[b]).wait()
                pltpu.async_copy(ones_v, acc_sh.at[idx_v.at[c]], sems[b],
                                 add=True)
            return carry

        lax.fori_loop(0, cpw // nb - 1, round_body, jnp.int32(0))
        for b in range(nb):
            pltpu.make_async_copy(ones_v, acc_sh.at[idx_v.at[cpw - nb + b]],
                                  sems[b]).wait()
        plsc.subcore_barrier()
        pltpu.sync_copy(acc_sh.at[pl.ds(start, sz)],
                        out_hbm.at[c_ax, pl.ds(start, sz)])

    return deg_kernel


@functools.lru_cache(maxsize=None)
def _agg_call(n, h, cpw, nc, ns):
    nw = nc * ns
    rpt, sz = _row_split(n, ns)
    nb = 8
    assert cpw % nb == 0 and cpw > nb

    @functools.partial(
        pl.kernel,
        out_type=jax.ShapeDtypeStruct((nc, n, h), jnp.float32),
        mesh=_mesh(),
        compiler_params=pltpu.CompilerParams(use_tc_tiling_on_sc=False),
        scratch_types=[
            pltpu.VMEM((cpw, _K), jnp.int32),
            pltpu.VMEM((cpw, _K), jnp.int32),
        ] + [pltpu.VMEM((_K, h), jnp.float32)] * nb + [
            pltpu.VMEM_SHARED((n + 8, h), jnp.float32),
        ] + [pltpu.SemaphoreType.DMA] * (2 * nb),
    )
    def agg_kernel(src_hbm, dst_hbm, s_hbm, zeros_hbm, out_hbm, *scr):
        idx_s, idx_d = scr[0], scr[1]
        rows = scr[2:2 + nb]
        acc_sh = scr[2 + nb]
        semg = scr[3 + nb:3 + 2 * nb]
        sems = scr[3 + 2 * nb:3 + 3 * nb]
        c_ax = lax.axis_index("c")
        s_ax = lax.axis_index("s")
        w = c_ax * ns + s_ax
        start = pl.multiple_of(s_ax * rpt // 8 * 8, 8)
        pltpu.sync_copy(src_hbm.at[pl.ds(w * cpw, cpw)], idx_s)
        pltpu.sync_copy(dst_hbm.at[pl.ds(w * cpw, cpw)], idx_d)
        pltpu.sync_copy(zeros_hbm.at[pl.ds(start, sz)],
                        acc_sh.at[pl.ds(start, sz)])
        plsc.subcore_barrier()
        for b in range(nb):
            pltpu.async_copy(s_hbm.at[idx_s.at[b]], rows[b], semg[b])

        def round_body(r, carry):
            base_c = r * nb
            descs = []
            for b in range(nb):
                c = base_c + b
                pltpu.make_async_copy(s_hbm.at[idx_s.at[c]], rows[b],
                                      semg[b]).wait()
                descs.append(pltpu.async_copy(
                    rows[b], acc_sh.at[idx_d.at[c]], sems[b], add=True))
            for b in range(nb):
                cn = jnp.minimum(base_c + nb + b, cpw - 1)
                descs[b].wait()
                pltpu.async_copy(s_hbm.at[idx_s.at[cn]], rows[b], semg[b])
            return carry

        lax.fori_loop(0, cpw // nb, round_body, jnp.int32(0))
        for b in range(nb):
            pltpu.make_async_copy(s_hbm.at[idx_s.at[cpw - 1]], rows[b],
                                  semg[b]).wait()
        plsc.subcore_barrier()
        pltpu.sync_copy(acc_sh.at[pl.ds(start, sz)],
                        out_hbm.at[c_ax, pl.ds(start, sz)])

    return agg_kernel


def _dot(a, b):
    return jnp.dot(a, b, precision=jax.lax.Precision.DEFAULT,
                   preferred_element_type=jnp.float32)


def _dot_exact(a, b):
    return jnp.dot(a, b, precision=jax.lax.Precision.HIGHEST,
                   preferred_element_type=jnp.float32)


def _bn_norm(agg, g, be):
    m = jnp.mean(agg, axis=0, keepdims=True)
    v = jnp.mean((agg - m) ** 2, axis=0, keepdims=True)
    return (agg - m) / jnp.sqrt(v + 1e-5) * g + be


def _tc1_body(degp_ref, x_ref, w1_ref, dinv_ref, s1_ref):
    deg = 1.0 + degp_ref[0, :, 0:1] + degp_ref[1, :, 0:1]
    dinv = lax.rsqrt(deg)
    dinv_ref[...] = dinv
    s1_ref[...] = _dot(x_ref[...], w1_ref[...]) * dinv


def _tc2_body(t1_ref, s1_ref, dinv_ref, b1_ref, g1_ref, be1_ref, w2_ref,
              h1_ref, s2_ref):
    dinv = dinv_ref[...]
    agg = dinv * (t1_ref[0] + t1_ref[1] + s1_ref[...]) + b1_ref[...]
    h1 = jnp.maximum(_bn_norm(agg, g1_ref[...], be1_ref[...]), 0.0)
    h1_ref[...] = h1
    s2_ref[...] = _dot(h1, w2_ref[...]) * dinv


def _tc3_body(t2_ref, s2_ref, dinv_ref, b2_ref, g2_ref, be2_ref, h1_ref,
              w3_ref, s3_ref):
    dinv = dinv_ref[...]
    agg = dinv * (t2_ref[0] + t2_ref[1] + s2_ref[...]) + b2_ref[...]
    h2 = jnp.maximum(_bn_norm(agg, g2_ref[...], be2_ref[...]) + h1_ref[...],
                     0.0)
    s3_ref[...] = _dot(h2, w3_ref[...]) * dinv


def _tc4_body(t3_ref, s3_ref, dinv_ref, b3_ref, g3_ref, be3_ref, batch_ref,
              sigma_ref, wf1_ref, bf1_ref, wf2_ref, bf2_ref, wfc_ref, bfc_ref,
              out_ref):
    gdim, n = out_ref.shape[0], s3_ref.shape[0]
    hdim = s3_ref.shape[1]
    dinv = dinv_ref[...]
    agg = dinv * (t3_ref[0] + t3_ref[1] + s3_ref[...]) + b3_ref[...]
    h3 = jnp.maximum(_bn_norm(agg, g3_ref[...], be3_ref[...]), 0.0)
    oh = (lax.broadcasted_iota(jnp.int32, (gdim, n), 0)
          == batch_ref[...]).astype(jnp.float32)
    sums = _dot_exact(oh, h3)
    cnt = jnp.sum(oh, axis=1, keepdims=True)
    gemb = sums / jnp.maximum(cnt, 1.0)
    f = jnp.maximum(_dot(sigma_ref[...], wf1_ref[...]) + bf1_ref[...], 0.0)
    f = jnp.maximum(_dot(f, wf2_ref[...]) + bf2_ref[...], 0.0)
    out_ref[...] = (_dot(gemb, wfc_ref[0:hdim, :])
                    + _dot(f, wfc_ref[hdim:, :]) + bfc_ref[...])


def kernel(x, edge_index, batch, sigma, W1, b1, W2, b2, W3, b3,
           g1, be1, g2, be2, g3, be3, Wf1, bf1, Wf2, bf2, Wfc, bfc):
    n, d = x.shape
    h = W1.shape[1]
    g = sigma.shape[0]
    e = edge_index.shape[1]
    info = plsc.get_sparse_core_info()
    nc, ns = info.num_cores, info.num_subcores
    nw = nc * ns

    nchunks = -(-e // _K)
    cpw = -(-(-(-nchunks // nw)) // 8) * 8
    pad = cpw * nw * _K - e
    src2d = jnp.concatenate(
        [edge_index[0], jnp.zeros((pad,), jnp.int32)]).reshape(-1, _K)
    dst2d = jnp.concatenate(
        [edge_index[1], jnp.full((pad,), n, jnp.int32)]).reshape(-1, _K)

    zeros_h = jnp.zeros((n, h), jnp.float32)
    zeros16 = jnp.zeros((n, 16), jnp.float32)
    ones_k16 = jnp.ones((_K, 16), jnp.float32)

    degp = _deg_call(n, cpw, nc, ns)(dst2d, ones_k16, zeros16)

    f32 = jnp.float32
    dinv, s1 = pl.pallas_call(
        _tc1_body,
        out_shape=[jax.ShapeDtypeStruct((n, 1), f32),
                   jax.ShapeDtypeStruct((n, h), f32)],
    )(degp, x, W1)

    agg = _agg_call(n, h, cpw, nc, ns)
    t1 = agg(src2d, dst2d, s1, zeros_h)
    h1, s2 = pl.pallas_call(
        _tc2_body,
        out_shape=[jax.ShapeDtypeStruct((n, h), f32),
                   jax.ShapeDtypeStruct((n, h), f32)],
    )(t1, s1, dinv, b1, g1, be1, W2)

    t2 = agg(src2d, dst2d, s2, zeros_h)
    s3 = pl.pallas_call(
        _tc3_body,
        out_shape=jax.ShapeDtypeStruct((n, h), f32),
    )(t2, s2, dinv, b2, g2, be2, h1, W3)

    t3 = agg(src2d, dst2d, s3, zeros_h)
    out2d = pl.pallas_call(
        _tc4_body,
        out_shape=jax.ShapeDtypeStruct((g, 1), f32),
    )(t3, s3, dinv, b3, g3, be3, batch.reshape(1, n), sigma,
      Wf1, bf1, Wf2, bf2, Wfc, bfc)
    return out2d.reshape(g)
